# Initial kernel scaffold; baseline (speedup 1.0000x reference)
import jax
import jax.numpy as jnp
from jax.experimental import pallas as pl


def kernel(cls_score, bbox_pred, dir_cls_pred, anchors):
    raise NotImplementedError("write your pallas kernel here")


# speedup vs baseline: 1.3398x; 1.3398x over previous
import functools

import jax
import jax.numpy as jnp
from jax import lax
from jax.experimental import pallas as pl
from jax.experimental.pallas import tpu as pltpu
from jax.experimental.pallas import tpu_sc as plsc

NUM_CLASSES = 3
BOX_CODE = 7
NMS_PRE = 1000
HW = 10000
HWP = 16384
K = 1024
LOGK = 10


_R = 256


def _stage(k, i, d, s):
    r_io = jax.lax.broadcasted_iota(jnp.int32, (_R, 128), 0)
    c_io = jax.lax.broadcasted_iota(jnp.int32, (_R, 128), 1)
    E = c_io * _R + r_io
    dirb = (jnp.zeros((_R, 128), jnp.bool_) if s is None
            else (((E >> s) & 1) == 1))
    if d >= _R:
        dc = d // _R
        mask_lo = ((c_io // dc) & 1) == 0
        ok = jnp.where(mask_lo, jnp.roll(k, -dc, axis=1),
                       jnp.roll(k, dc, axis=1))
        oi = jnp.where(mask_lo, jnp.roll(i, -dc, axis=1),
                       jnp.roll(i, dc, axis=1))
        cmp = (k > ok) | ((k == ok) & (i < oi))
        keep = (cmp != (~mask_lo)) != dirb
        return jnp.where(keep, k, ok), jnp.where(keep, i, oi)
    dr = d
    R2 = _R // (2 * dr)
    ks = k.reshape(R2, 2, dr, 128)
    js = i.reshape(R2, 2, dr, 128)
    ds_ = dirb.reshape(R2, 2, dr, 128)[:, 0]
    ak, bk = ks[:, 0], ks[:, 1]
    ai, bi = js[:, 0], js[:, 1]
    swap = ((ak < bk) | ((ak == bk) & (ai > bi))) != ds_
    nak = jnp.where(swap, bk, ak)
    nbk = jnp.where(swap, ak, bk)
    nai = jnp.where(swap, bi, ai)
    nbi = jnp.where(swap, ai, bi)
    nk = jnp.stack([nak, nbk], axis=1).reshape(_R, 128)
    ni = jnp.stack([nai, nbi], axis=1).reshape(_R, 128)
    return nk, ni


def _row_stage_dyn(k, i, jd, s):
    r_io = jax.lax.broadcasted_iota(jnp.int32, (_R, 128), 0)
    c_io = jax.lax.broadcasted_iota(jnp.int32, (_R, 128), 1)
    E = c_io * _R + r_io
    dirb = (jnp.zeros((_R, 128), jnp.bool_) if s is None
            else (((E >> s) & 1) == 1))
    d = jnp.int32(1) << jd
    mask_lo = ((E >> jd) & 1) == 0
    ok = jnp.where(mask_lo, pltpu.roll(k, _R - d, axis=0),
                   pltpu.roll(k, d, axis=0))
    oi = jnp.where(mask_lo, pltpu.roll(i, _R - d, axis=0),
                   pltpu.roll(i, d, axis=0))
    cmp = (k > ok) | ((k == ok) & (i < oi))
    keep = (cmp != (~mask_lo)) != dirb
    return jnp.where(keep, k, ok), jnp.where(keep, i, oi)


def _topk_body(cls_ref, out_ref):
    cls = cls_ref[...]
    m = jnp.max(cls, axis=1)
    p = jax.lax.broadcasted_iota(jnp.int32, (2, HWP), 1)
    a = jax.lax.broadcasted_iota(jnp.int32, (2, HWP), 0)
    k = m.reshape(_R, 128)
    i = (2 * p + a).reshape(_R, 128)

    def row_fori(k, i, jhi, n, s):
        def body(u, ki):
            return _row_stage_dyn(ki[0], ki[1], jhi - u, s)
        return jax.lax.fori_loop(0, n, body, (k, i))

    for s in range(1, LOGK + 1):
        for j in range(s - 1, 7, -1):
            k, i = _stage(k, i, 1 << j, s)
        k, i = row_fori(k, i, min(s - 1, 7), min(s, 8), s)
    for t in range(5):
        k, i = _stage(k, i, K << t, None)
        k, i = _stage(k, i, 512, LOGK + 1 + t)
        k, i = _stage(k, i, 256, LOGK + 1 + t)
        k, i = row_fori(k, i, 7, 8, LOGK + 1 + t)
    out_ref[...] = i[:, 0:4]


@functools.partial(jax.jit, static_argnames=("interpret",))
def _topk_call(clsp, interpret=False):
    return pl.pallas_call(
        _topk_body,
        out_shape=jax.ShapeDtypeStruct((_R, 4), jnp.int32),
        interpret=interpret,
    )(clsp)


def _sigmoid(x):
    return 1.0 / (1.0 + jnp.exp(-x))


def _sqrt_sc(x):
    ii = lax.bitcast_convert_type(x, jnp.int32)
    y = lax.bitcast_convert_type(jnp.int32(0x5F3759DF) - (ii >> 1), jnp.float32)
    for _ in range(3):
        y = y * (1.5 - 0.5 * x * y * y)
    return x * y


def _sc_tail_call(inds, cls_flat, bbox_flat, dir_flat, anc_flat):
    NC, NS = 2, 16
    NW = NC * NS
    B = K // NW
    HV = B // 16
    mesh = plsc.VectorSubcoreMesh(core_axis_name="c", subcore_axis_name="s")

    @functools.partial(
        pl.kernel, mesh=mesh,
        out_type=[
            jax.ShapeDtypeStruct((NUM_CLASSES, K), jnp.float32),
            jax.ShapeDtypeStruct((BOX_CODE, K), jnp.float32),
            jax.ShapeDtypeStruct((K,), jnp.int32),
        ],
        scratch_types=[
            pltpu.VMEM((B,), jnp.int32),
            pltpu.VMEM((BOX_CODE, B), jnp.int32),
            pltpu.VMEM((BOX_CODE, B), jnp.int32),
            pltpu.VMEM((NUM_CLASSES, B), jnp.int32),
            pltpu.VMEM((2, B), jnp.int32),
            pltpu.VMEM((BOX_CODE, B), jnp.float32),
            pltpu.VMEM((BOX_CODE, B), jnp.float32),
            pltpu.VMEM((NUM_CLASSES, B), jnp.float32),
            pltpu.VMEM((2, B), jnp.float32),
            pltpu.VMEM((NUM_CLASSES, B), jnp.float32),
            pltpu.VMEM((BOX_CODE, B), jnp.float32),
            pltpu.VMEM((B,), jnp.int32),
            pltpu.SemaphoreType.DMA,
        ],
    )
    def tail(inds_hbm, cls_hbm, bbox_hbm, dir_hbm, anc_hbm,
             scores_out, bbox_out, dir_out,
             inds_v, ai_v, bi_v, ci_v, di_v, av_v, bv_v, cv_v, dv_v,
             so_v, bo_v, do_v, sem):
        wid = lax.axis_index("s") * NC + lax.axis_index("c")
        base = wid * B
        pltpu.sync_copy(inds_hbm.at[pl.ds(base, B)], inds_v)
        for h in range(HV):
            n = inds_v[pl.ds(h * 16, 16)]
            pa = n >> 1
            aa = n & 1
            for kk in range(BOX_CODE):
                ai_v[kk, pl.ds(h * 16, 16)] = n * BOX_CODE + kk
                bi_v[kk, pl.ds(h * 16, 16)] = (aa * BOX_CODE + kk) * HW + pa
            for cc in range(NUM_CLASSES):
                ci_v[cc, pl.ds(h * 16, 16)] = (aa * NUM_CLASSES + cc) * HW + pa
            for dd in range(2):
                di_v[dd, pl.ds(h * 16, 16)] = (aa * 2 + dd) * HW + pa
        copies = []
        for kk in range(BOX_CODE):
            copies.append(pltpu.async_copy(anc_hbm.at[ai_v.at[kk]],
                                           av_v.at[kk], sem))
            copies.append(pltpu.async_copy(bbox_hbm.at[bi_v.at[kk]],
                                           bv_v.at[kk], sem))
        for cc in range(NUM_CLASSES):
            copies.append(pltpu.async_copy(cls_hbm.at[ci_v.at[cc]],
                                           cv_v.at[cc], sem))
        for dd in range(2):
            copies.append(pltpu.async_copy(dir_hbm.at[di_v.at[dd]],
                                           dv_v.at[dd], sem))
        for cp in copies:
            cp.wait()
        for h in range(HV):
            hs = pl.ds(h * 16, 16)
            xa, ya, za, wa, la, ha, ra = (av_v[kk, hs] for kk in range(7))
            xt, yt, zt, wt, lt, ht, rt = (bv_v[kk, hs] for kk in range(7))
            za = za + ha * 0.5
            diag = _sqrt_sc(la * la + wa * wa)
            hg = jnp.exp(ht) * ha
            comps = (
                xt * diag + xa,
                yt * diag + ya,
                zt * ha + za - hg * 0.5,
                jnp.exp(wt) * wa,
                jnp.exp(lt) * la,
                hg,
                rt + ra,
            )
            for kk in range(BOX_CODE):
                bo_v[kk, hs] = comps[kk]
            for cc in range(NUM_CLASSES):
                so_v[cc, hs] = _sigmoid(cv_v[cc, hs])
            do_v[hs] = jnp.where(dv_v[1, hs] > dv_v[0, hs],
                                 jnp.int32(1), jnp.int32(0))
        for cc in range(NUM_CLASSES):
            pltpu.sync_copy(so_v.at[cc], scores_out.at[cc, pl.ds(base, B)])
        for kk in range(BOX_CODE):
            pltpu.sync_copy(bo_v.at[kk], bbox_out.at[kk, pl.ds(base, B)])
        pltpu.sync_copy(do_v, dir_out.at[pl.ds(base, B)])

    return tail(inds, cls_flat, bbox_flat, dir_flat, anc_flat)


def _decode(anchors, deltas):
    xa, ya, za, wa, la, ha, ra = jnp.split(anchors, 7, axis=-1)
    xt, yt, zt, wt, lt, ht, rt = jnp.split(deltas, 7, axis=-1)
    za = za + ha / 2
    diagonal = jnp.sqrt(la ** 2 + wa ** 2)
    xg = xt * diagonal + xa
    yg = yt * diagonal + ya
    zg = zt * ha + za
    lg = jnp.exp(lt) * la
    wg = jnp.exp(wt) * wa
    hg = jnp.exp(ht) * ha
    rg = rt + ra
    zg = zg - hg / 2
    return jnp.concatenate([xg, yg, zg, wg, lg, hg, rg], axis=-1)


def kernel(cls_score, bbox_pred, dir_cls_pred, anchors, _interpret=False):
    cls3 = cls_score.reshape(2, NUM_CLASSES, HW)
    clsp = jnp.pad(cls3, ((0, 0), (0, 0), (0, HWP - HW)),
                   constant_values=-jnp.inf)
    inds1024 = jnp.transpose(_topk_call(clsp, interpret=_interpret)).reshape(K)
    if _interpret:
        topk_inds = inds1024[:NMS_PRE]
        dir_flat = jnp.transpose(dir_cls_pred, (1, 2, 0)).reshape(-1, 2)
        dir_cls_scores = jnp.argmax(dir_flat, axis=-1)
        cls = jax.nn.sigmoid(
            jnp.transpose(cls_score, (1, 2, 0)).reshape(-1, NUM_CLASSES))
        bbox = jnp.transpose(bbox_pred, (1, 2, 0)).reshape(-1, BOX_CODE)
        anchors_ = anchors[topk_inds, :]
        bbox_ = bbox[topk_inds, :]
        scores = cls[topk_inds, :]
        dir_cls_score = dir_cls_scores[topk_inds]
        bboxes = _decode(anchors_, bbox_)
        return (scores, bboxes, dir_cls_score)
    scores_f, bbox_f, dir_f = _sc_tail_call(
        inds1024,
        cls_score.reshape(-1),
        bbox_pred.reshape(-1),
        dir_cls_pred.reshape(-1),
        anchors.reshape(-1),
    )
    scores = jnp.transpose(scores_f)[:NMS_PRE]
    bboxes = jnp.transpose(bbox_f)[:NMS_PRE]
    dir_cls_score = dir_f[:NMS_PRE]
    return (scores, bboxes, dir_cls_score)


# speedup vs baseline: 1.9049x; 1.4218x over previous
import functools

import jax
import jax.numpy as jnp
from jax import lax
from jax.experimental import pallas as pl
from jax.experimental.pallas import tpu as pltpu
from jax.experimental.pallas import tpu_sc as plsc

NUM_CLASSES = 3
BOX_CODE = 7
NMS_PRE = 1000
HW = 10000
HWP = 16384
K = 1024
LOGK = 10


_R = 256


def _stage(k, i, d, s):
    r_io = jax.lax.broadcasted_iota(jnp.int32, (_R, 128), 0)
    c_io = jax.lax.broadcasted_iota(jnp.int32, (_R, 128), 1)
    E = c_io * _R + r_io
    dirb = (jnp.zeros((_R, 128), jnp.bool_) if s is None
            else (((E >> s) & 1) == 1))
    if d >= _R:
        dc = d // _R
        mask_lo = ((c_io // dc) & 1) == 0
        ok = jnp.where(mask_lo, jnp.roll(k, -dc, axis=1),
                       jnp.roll(k, dc, axis=1))
        oi = jnp.where(mask_lo, jnp.roll(i, -dc, axis=1),
                       jnp.roll(i, dc, axis=1))
        cmp = (k > ok) | ((k == ok) & (i < oi))
        keep = (cmp != (~mask_lo)) != dirb
        return jnp.where(keep, k, ok), jnp.where(keep, i, oi)
    dr = d
    R2 = _R // (2 * dr)
    ks = k.reshape(R2, 2, dr, 128)
    js = i.reshape(R2, 2, dr, 128)
    ds_ = dirb.reshape(R2, 2, dr, 128)[:, 0]
    ak, bk = ks[:, 0], ks[:, 1]
    ai, bi = js[:, 0], js[:, 1]
    swap = ((ak < bk) | ((ak == bk) & (ai > bi))) != ds_
    nak = jnp.where(swap, bk, ak)
    nbk = jnp.where(swap, ak, bk)
    nai = jnp.where(swap, bi, ai)
    nbi = jnp.where(swap, ai, bi)
    nk = jnp.stack([nak, nbk], axis=1).reshape(_R, 128)
    ni = jnp.stack([nai, nbi], axis=1).reshape(_R, 128)
    return nk, ni


def _row_stage(k, i, d, s):
    r_io = jax.lax.broadcasted_iota(jnp.int32, (_R, 128), 0)
    c_io = jax.lax.broadcasted_iota(jnp.int32, (_R, 128), 1)
    E = c_io * _R + r_io
    dirb = (jnp.zeros((_R, 128), jnp.bool_) if s is None
            else (((E >> s) & 1) == 1))
    mask_lo = (r_io & d) == 0
    ok = jnp.where(mask_lo, jnp.roll(k, -d, axis=0), jnp.roll(k, d, axis=0))
    oi = jnp.where(mask_lo, jnp.roll(i, -d, axis=0), jnp.roll(i, d, axis=0))
    cmp = (k > ok) | ((k == ok) & (i < oi))
    keep = (cmp != (~mask_lo)) != dirb
    return jnp.where(keep, k, ok), jnp.where(keep, i, oi)


def _topk_body(cls_ref, out_ref):
    cls = cls_ref[...]
    m = jnp.max(cls, axis=1)
    p = jax.lax.broadcasted_iota(jnp.int32, (2, HWP), 1)
    a = jax.lax.broadcasted_iota(jnp.int32, (2, HWP), 0)
    k = m.reshape(_R, 128)
    i = (2 * p + a).reshape(_R, 128)

    def any_stage(k, i, d, s):
        if d >= _R:
            return _stage(k, i, d, s)
        return _row_stage(k, i, d, s)

    for s in range(1, LOGK + 1):
        for j in range(s - 1, -1, -1):
            k, i = any_stage(k, i, 1 << j, s)
    for t in range(5):
        k, i = any_stage(k, i, K << t, None)
        for j in range(LOGK - 1, -1, -1):
            k, i = any_stage(k, i, 1 << j, LOGK + 1 + t)
    out_ref[...] = i[:, 0:4]


@functools.partial(jax.jit, static_argnames=("interpret",))
def _topk_call(clsp, interpret=False):
    return pl.pallas_call(
        _topk_body,
        out_shape=jax.ShapeDtypeStruct((_R, 4), jnp.int32),
        interpret=interpret,
    )(clsp)


def _sigmoid(x):
    return 1.0 / (1.0 + jnp.exp(-x))


def _sqrt_sc(x):
    ii = lax.bitcast_convert_type(x, jnp.int32)
    y = lax.bitcast_convert_type(jnp.int32(0x5F3759DF) - (ii >> 1), jnp.float32)
    for _ in range(3):
        y = y * (1.5 - 0.5 * x * y * y)
    return x * y


def _sc_tail_call(inds, cls_flat, bbox_flat, dir_flat, anc_flat):
    NC, NS = 2, 16
    NW = NC * NS
    B = K // NW
    HV = B // 16
    mesh = plsc.VectorSubcoreMesh(core_axis_name="c", subcore_axis_name="s")

    @functools.partial(
        pl.kernel, mesh=mesh,
        out_type=[
            jax.ShapeDtypeStruct((NUM_CLASSES, K), jnp.float32),
            jax.ShapeDtypeStruct((BOX_CODE, K), jnp.float32),
            jax.ShapeDtypeStruct((K,), jnp.int32),
        ],
        scratch_types=[
            pltpu.VMEM((B,), jnp.int32),
            pltpu.VMEM((BOX_CODE, B), jnp.int32),
            pltpu.VMEM((BOX_CODE, B), jnp.int32),
            pltpu.VMEM((NUM_CLASSES, B), jnp.int32),
            pltpu.VMEM((2, B), jnp.int32),
            pltpu.VMEM((BOX_CODE, B), jnp.float32),
            pltpu.VMEM((BOX_CODE, B), jnp.float32),
            pltpu.VMEM((NUM_CLASSES, B), jnp.float32),
            pltpu.VMEM((2, B), jnp.float32),
            pltpu.VMEM((NUM_CLASSES, B), jnp.float32),
            pltpu.VMEM((BOX_CODE, B), jnp.float32),
            pltpu.VMEM((B,), jnp.int32),
            pltpu.SemaphoreType.DMA,
        ],
    )
    def tail(inds_hbm, cls_hbm, bbox_hbm, dir_hbm, anc_hbm,
             scores_out, bbox_out, dir_out,
             inds_v, ai_v, bi_v, ci_v, di_v, av_v, bv_v, cv_v, dv_v,
             so_v, bo_v, do_v, sem):
        wid = lax.axis_index("s") * NC + lax.axis_index("c")
        base = wid * B
        pltpu.sync_copy(inds_hbm.at[pl.ds(base, B)], inds_v)
        for h in range(HV):
            n = inds_v[pl.ds(h * 16, 16)]
            pa = n >> 1
            aa = n & 1
            for kk in range(BOX_CODE):
                ai_v[kk, pl.ds(h * 16, 16)] = n * BOX_CODE + kk
                bi_v[kk, pl.ds(h * 16, 16)] = (aa * BOX_CODE + kk) * HW + pa
            for cc in range(NUM_CLASSES):
                ci_v[cc, pl.ds(h * 16, 16)] = (aa * NUM_CLASSES + cc) * HW + pa
            for dd in range(2):
                di_v[dd, pl.ds(h * 16, 16)] = (aa * 2 + dd) * HW + pa
        copies = []
        for kk in range(BOX_CODE):
            copies.append(pltpu.async_copy(anc_hbm.at[ai_v.at[kk]],
                                           av_v.at[kk], sem))
            copies.append(pltpu.async_copy(bbox_hbm.at[bi_v.at[kk]],
                                           bv_v.at[kk], sem))
        for cc in range(NUM_CLASSES):
            copies.append(pltpu.async_copy(cls_hbm.at[ci_v.at[cc]],
                                           cv_v.at[cc], sem))
        for dd in range(2):
            copies.append(pltpu.async_copy(dir_hbm.at[di_v.at[dd]],
                                           dv_v.at[dd], sem))
        for cp in copies:
            cp.wait()
        for h in range(HV):
            hs = pl.ds(h * 16, 16)
            xa, ya, za, wa, la, ha, ra = (av_v[kk, hs] for kk in range(7))
            xt, yt, zt, wt, lt, ht, rt = (bv_v[kk, hs] for kk in range(7))
            za = za + ha * 0.5
            diag = _sqrt_sc(la * la + wa * wa)
            hg = jnp.exp(ht) * ha
            comps = (
                xt * diag + xa,
                yt * diag + ya,
                zt * ha + za - hg * 0.5,
                jnp.exp(wt) * wa,
                jnp.exp(lt) * la,
                hg,
                rt + ra,
            )
            for kk in range(BOX_CODE):
                bo_v[kk, hs] = comps[kk]
            for cc in range(NUM_CLASSES):
                so_v[cc, hs] = _sigmoid(cv_v[cc, hs])
            do_v[hs] = jnp.where(dv_v[1, hs] > dv_v[0, hs],
                                 jnp.int32(1), jnp.int32(0))
        for cc in range(NUM_CLASSES):
            pltpu.sync_copy(so_v.at[cc], scores_out.at[cc, pl.ds(base, B)])
        for kk in range(BOX_CODE):
            pltpu.sync_copy(bo_v.at[kk], bbox_out.at[kk, pl.ds(base, B)])
        pltpu.sync_copy(do_v, dir_out.at[pl.ds(base, B)])

    return tail(inds, cls_flat, bbox_flat, dir_flat, anc_flat)


def _decode(anchors, deltas):
    xa, ya, za, wa, la, ha, ra = jnp.split(anchors, 7, axis=-1)
    xt, yt, zt, wt, lt, ht, rt = jnp.split(deltas, 7, axis=-1)
    za = za + ha / 2
    diagonal = jnp.sqrt(la ** 2 + wa ** 2)
    xg = xt * diagonal + xa
    yg = yt * diagonal + ya
    zg = zt * ha + za
    lg = jnp.exp(lt) * la
    wg = jnp.exp(wt) * wa
    hg = jnp.exp(ht) * ha
    rg = rt + ra
    zg = zg - hg / 2
    return jnp.concatenate([xg, yg, zg, wg, lg, hg, rg], axis=-1)


def kernel(cls_score, bbox_pred, dir_cls_pred, anchors, _interpret=False):
    cls3 = cls_score.reshape(2, NUM_CLASSES, HW)
    clsp = jnp.pad(cls3, ((0, 0), (0, 0), (0, HWP - HW)),
                   constant_values=-jnp.inf)
    inds1024 = jnp.transpose(_topk_call(clsp, interpret=_interpret)).reshape(K)
    if _interpret:
        topk_inds = inds1024[:NMS_PRE]
        dir_flat = jnp.transpose(dir_cls_pred, (1, 2, 0)).reshape(-1, 2)
        dir_cls_scores = jnp.argmax(dir_flat, axis=-1)
        cls = jax.nn.sigmoid(
            jnp.transpose(cls_score, (1, 2, 0)).reshape(-1, NUM_CLASSES))
        bbox = jnp.transpose(bbox_pred, (1, 2, 0)).reshape(-1, BOX_CODE)
        anchors_ = anchors[topk_inds, :]
        bbox_ = bbox[topk_inds, :]
        scores = cls[topk_inds, :]
        dir_cls_score = dir_cls_scores[topk_inds]
        bboxes = _decode(anchors_, bbox_)
        return (scores, bboxes, dir_cls_score)
    scores_f, bbox_f, dir_f = _sc_tail_call(
        inds1024,
        cls_score.reshape(-1),
        bbox_pred.reshape(-1),
        dir_cls_pred.reshape(-1),
        anchors.reshape(-1),
    )
    scores = jnp.transpose(scores_f)[:NMS_PRE]
    bboxes = jnp.transpose(bbox_f)[:NMS_PRE]
    dir_cls_score = dir_f[:NMS_PRE]
    return (scores, bboxes, dir_cls_score)


# speedup vs baseline: 2.0562x; 1.0794x over previous
import functools

import jax
import jax.numpy as jnp
from jax import lax
from jax.experimental import pallas as pl
from jax.experimental.pallas import tpu as pltpu
from jax.experimental.pallas import tpu_sc as plsc

NUM_CLASSES = 3
BOX_CODE = 7
NMS_PRE = 1000
HW = 10000
HWP = 16384
K = 1024
LOGK = 10


_R = 256


def _stage(k, i, d, s):
    r_io = jax.lax.broadcasted_iota(jnp.int32, (_R, 128), 0)
    c_io = jax.lax.broadcasted_iota(jnp.int32, (_R, 128), 1)
    E = c_io * _R + r_io
    dirb = (jnp.zeros((_R, 128), jnp.bool_) if s is None
            else (((E >> s) & 1) == 1))
    if d >= _R:
        dc = d // _R
        mask_lo = ((c_io // dc) & 1) == 0
        ok = jnp.where(mask_lo, jnp.roll(k, -dc, axis=1),
                       jnp.roll(k, dc, axis=1))
        oi = jnp.where(mask_lo, jnp.roll(i, -dc, axis=1),
                       jnp.roll(i, dc, axis=1))
        cmp = (k > ok) | ((k == ok) & (i < oi))
        keep = (cmp != (~mask_lo)) != dirb
        return jnp.where(keep, k, ok), jnp.where(keep, i, oi)
    dr = d
    R2 = _R // (2 * dr)
    ks = k.reshape(R2, 2, dr, 128)
    js = i.reshape(R2, 2, dr, 128)
    ds_ = dirb.reshape(R2, 2, dr, 128)[:, 0]
    ak, bk = ks[:, 0], ks[:, 1]
    ai, bi = js[:, 0], js[:, 1]
    swap = ((ak < bk) | ((ak == bk) & (ai > bi))) != ds_
    nak = jnp.where(swap, bk, ak)
    nbk = jnp.where(swap, ak, bk)
    nai = jnp.where(swap, bi, ai)
    nbi = jnp.where(swap, ai, bi)
    nk = jnp.stack([nak, nbk], axis=1).reshape(_R, 128)
    ni = jnp.stack([nai, nbi], axis=1).reshape(_R, 128)
    return nk, ni


def _row_stage(k, i, d, s):
    r_io = jax.lax.broadcasted_iota(jnp.int32, (_R, 128), 0)
    c_io = jax.lax.broadcasted_iota(jnp.int32, (_R, 128), 1)
    E = c_io * _R + r_io
    dirb = (jnp.zeros((_R, 128), jnp.bool_) if s is None
            else (((E >> s) & 1) == 1))
    mask_lo = (r_io & d) == 0
    ok = jnp.where(mask_lo, jnp.roll(k, -d, axis=0), jnp.roll(k, d, axis=0))
    oi = jnp.where(mask_lo, jnp.roll(i, -d, axis=0), jnp.roll(i, d, axis=0))
    cmp = (k > ok) | ((k == ok) & (i < oi))
    keep = (cmp != (~mask_lo)) != dirb
    return jnp.where(keep, k, ok), jnp.where(keep, i, oi)


def _topk_body(cls_ref, out_ref):
    cls = cls_ref[...]
    m0 = jnp.max(cls[0:3], axis=0)
    m1 = jnp.max(cls[3:6], axis=0)
    neg = jnp.float32(-jnp.inf)
    k2 = jnp.concatenate([m0, m1], axis=0)
    k2 = jnp.concatenate([k2, jnp.full((_R - 200, 100), neg)], axis=0)
    k = jnp.concatenate([k2, jnp.full((_R, 28), neg)], axis=1)
    r_io = jax.lax.broadcasted_iota(jnp.int32, (_R, 128), 0)
    c_io = jax.lax.broadcasted_iota(jnp.int32, (_R, 128), 1)
    h = jnp.where(r_io < 100, r_io, r_io - 100)
    a = jnp.where(r_io < 100, 0, 1)
    valid = (r_io < 200) & (c_io < 100)
    i = jnp.where(valid, 2 * (h * 100 + c_io) + a,
                  2_000_000 + r_io * 128 + c_io)

    def any_stage(k, i, d, s):
        if d >= _R:
            return _stage(k, i, d, s)
        return _row_stage(k, i, d, s)

    for s in range(1, LOGK + 1):
        for j in range(s - 1, -1, -1):
            k, i = any_stage(k, i, 1 << j, s)
    for t in range(5):
        k, i = any_stage(k, i, K << t, None)
        for j in range(LOGK - 1, -1, -1):
            k, i = any_stage(k, i, 1 << j, LOGK + 1 + t)
    out_ref[...] = jnp.transpose(i[:, 0:4])


@functools.partial(jax.jit, static_argnames=("interpret",))
def _topk_call(cls_score, interpret=False):
    return pl.pallas_call(
        _topk_body,
        out_shape=jax.ShapeDtypeStruct((4, _R), jnp.int32),
        interpret=interpret,
    )(cls_score)


def _sigmoid(x):
    return 1.0 / (1.0 + jnp.exp(-x))


def _sqrt_sc(x):
    ii = lax.bitcast_convert_type(x, jnp.int32)
    y = lax.bitcast_convert_type(jnp.int32(0x5F3759DF) - (ii >> 1), jnp.float32)
    for _ in range(3):
        y = y * (1.5 - 0.5 * x * y * y)
    return x * y


def _sc_tail_call(inds, cls_flat, bbox_flat, dir_flat, anc_flat):
    NC, NS = 2, 16
    NW = NC * NS
    B = K // NW
    HV = B // 16
    mesh = plsc.VectorSubcoreMesh(core_axis_name="c", subcore_axis_name="s")

    @functools.partial(
        pl.kernel, mesh=mesh,
        out_type=[
            jax.ShapeDtypeStruct((NUM_CLASSES, K), jnp.float32),
            jax.ShapeDtypeStruct((BOX_CODE, K), jnp.float32),
            jax.ShapeDtypeStruct((K,), jnp.int32),
        ],
        scratch_types=[
            pltpu.VMEM((B,), jnp.int32),
            pltpu.VMEM((BOX_CODE, B), jnp.int32),
            pltpu.VMEM((BOX_CODE, B), jnp.int32),
            pltpu.VMEM((NUM_CLASSES, B), jnp.int32),
            pltpu.VMEM((2, B), jnp.int32),
            pltpu.VMEM((BOX_CODE, B), jnp.float32),
            pltpu.VMEM((BOX_CODE, B), jnp.float32),
            pltpu.VMEM((NUM_CLASSES, B), jnp.float32),
            pltpu.VMEM((2, B), jnp.float32),
            pltpu.VMEM((NUM_CLASSES, B), jnp.float32),
            pltpu.VMEM((BOX_CODE, B), jnp.float32),
            pltpu.VMEM((B,), jnp.int32),
            pltpu.SemaphoreType.DMA,
        ],
    )
    def tail(inds_hbm, cls_hbm, bbox_hbm, dir_hbm, anc_hbm,
             scores_out, bbox_out, dir_out,
             inds_v, ai_v, bi_v, ci_v, di_v, av_v, bv_v, cv_v, dv_v,
             so_v, bo_v, do_v, sem):
        wid = lax.axis_index("s") * NC + lax.axis_index("c")
        base = wid * B
        pltpu.sync_copy(inds_hbm.at[wid // 8, pl.ds((wid % 8) * B, B)],
                        inds_v)
        for h in range(HV):
            n = inds_v[pl.ds(h * 16, 16)]
            pa = n >> 1
            aa = n & 1
            for kk in range(BOX_CODE):
                ai_v[kk, pl.ds(h * 16, 16)] = n * BOX_CODE + kk
                bi_v[kk, pl.ds(h * 16, 16)] = (aa * BOX_CODE + kk) * HW + pa
            for cc in range(NUM_CLASSES):
                ci_v[cc, pl.ds(h * 16, 16)] = (aa * NUM_CLASSES + cc) * HW + pa
            for dd in range(2):
                di_v[dd, pl.ds(h * 16, 16)] = (aa * 2 + dd) * HW + pa
        copies = []
        for kk in range(BOX_CODE):
            copies.append(pltpu.async_copy(anc_hbm.at[ai_v.at[kk]],
                                           av_v.at[kk], sem))
            copies.append(pltpu.async_copy(bbox_hbm.at[bi_v.at[kk]],
                                           bv_v.at[kk], sem))
        for cc in range(NUM_CLASSES):
            copies.append(pltpu.async_copy(cls_hbm.at[ci_v.at[cc]],
                                           cv_v.at[cc], sem))
        for dd in range(2):
            copies.append(pltpu.async_copy(dir_hbm.at[di_v.at[dd]],
                                           dv_v.at[dd], sem))
        for cp in copies:
            cp.wait()
        for h in range(HV):
            hs = pl.ds(h * 16, 16)
            xa, ya, za, wa, la, ha, ra = (av_v[kk, hs] for kk in range(7))
            xt, yt, zt, wt, lt, ht, rt = (bv_v[kk, hs] for kk in range(7))
            za = za + ha * 0.5
            diag = _sqrt_sc(la * la + wa * wa)
            hg = jnp.exp(ht) * ha
            comps = (
                xt * diag + xa,
                yt * diag + ya,
                zt * ha + za - hg * 0.5,
                jnp.exp(wt) * wa,
                jnp.exp(lt) * la,
                hg,
                rt + ra,
            )
            for kk in range(BOX_CODE):
                bo_v[kk, hs] = comps[kk]
            for cc in range(NUM_CLASSES):
                so_v[cc, hs] = _sigmoid(cv_v[cc, hs])
            do_v[hs] = jnp.where(dv_v[1, hs] > dv_v[0, hs],
                                 jnp.int32(1), jnp.int32(0))
        for cc in range(NUM_CLASSES):
            pltpu.sync_copy(so_v.at[cc], scores_out.at[cc, pl.ds(base, B)])
        for kk in range(BOX_CODE):
            pltpu.sync_copy(bo_v.at[kk], bbox_out.at[kk, pl.ds(base, B)])
        pltpu.sync_copy(do_v, dir_out.at[pl.ds(base, B)])

    return tail(inds, cls_flat, bbox_flat, dir_flat, anc_flat)


def _decode(anchors, deltas):
    xa, ya, za, wa, la, ha, ra = jnp.split(anchors, 7, axis=-1)
    xt, yt, zt, wt, lt, ht, rt = jnp.split(deltas, 7, axis=-1)
    za = za + ha / 2
    diagonal = jnp.sqrt(la ** 2 + wa ** 2)
    xg = xt * diagonal + xa
    yg = yt * diagonal + ya
    zg = zt * ha + za
    lg = jnp.exp(lt) * la
    wg = jnp.exp(wt) * wa
    hg = jnp.exp(ht) * ha
    rg = rt + ra
    zg = zg - hg / 2
    return jnp.concatenate([xg, yg, zg, wg, lg, hg, rg], axis=-1)


def kernel(cls_score, bbox_pred, dir_cls_pred, anchors, _interpret=False):
    inds4 = _topk_call(cls_score, interpret=_interpret)
    if _interpret:
        topk_inds = inds4.reshape(K)[:NMS_PRE]
        dir_flat = jnp.transpose(dir_cls_pred, (1, 2, 0)).reshape(-1, 2)
        dir_cls_scores = jnp.argmax(dir_flat, axis=-1)
        cls = jax.nn.sigmoid(
            jnp.transpose(cls_score, (1, 2, 0)).reshape(-1, NUM_CLASSES))
        bbox = jnp.transpose(bbox_pred, (1, 2, 0)).reshape(-1, BOX_CODE)
        anchors_ = anchors[topk_inds, :]
        bbox_ = bbox[topk_inds, :]
        scores = cls[topk_inds, :]
        dir_cls_score = dir_cls_scores[topk_inds]
        bboxes = _decode(anchors_, bbox_)
        return (scores, bboxes, dir_cls_score)
    scores_f, bbox_f, dir_f = _sc_tail_call(
        inds4,
        cls_score.reshape(-1),
        bbox_pred.reshape(-1),
        dir_cls_pred.reshape(-1),
        anchors.reshape(-1),
    )
    scores = jnp.transpose(scores_f)[:NMS_PRE]
    bboxes = jnp.transpose(bbox_f)[:NMS_PRE]
    dir_cls_score = dir_f[:NMS_PRE]
    return (scores, bboxes, dir_cls_score)
